# baseline (speedup 1.0000x reference)
import jax
import jax.numpy as jnp
from jax import lax
from jax.experimental import pallas as pl
from jax.experimental.pallas import tpu as pltpu
from jax.experimental.pallas import tpu_sc as plsc

NC = 2
NS = 16
NW = NC * NS
L = 16

B = 4
M = 8192
N = B * M
PPW = N // NW
CHUNK = 16
NCHUNK = PPW // CHUNK

C0 = 32
S0, S1 = 64, 32
CAP0 = float(S0 - 1.01)
CAP1 = float(S1 - 1.01)
LO = 0.01


def _axis(p, scale, cap):
    t = jnp.clip(p * scale, LO, cap)
    i1 = t.astype(jnp.int32)
    f1 = i1.astype(jnp.float32)
    i2 = i1 + jnp.where(t != f1, 1, 0)
    f2 = i2.astype(jnp.float32)
    return i1, i2, t - f1, f2 - t


def _corner_rows(px, py, pz, scale, cap, bbase, s):
    x1, x2, _, _ = _axis(px, scale, cap)
    y1, y2, _, _ = _axis(py, scale, cap)
    z1, z2, _, _ = _axis(pz, scale, cap)
    s2 = s * s
    ax = (bbase + x1 * s2, bbase + x2 * s2)
    ay = (y1 * s, y2 * s)
    az = (z1, z2)
    rows = []
    for dx in range(2):
        for dy in range(2):
            axy = ax[dx] + ay[dy]
            for dz in range(2):
                rows.append(axy + az[dz])
    return rows


def _corner_weights(px, py, pz, scale, cap):
    _, _, wx, wx2 = _axis(px, scale, cap)
    _, _, wy, wy2 = _axis(py, scale, cap)
    _, _, wz, wz2 = _axis(pz, scale, cap)
    wxs = (wx2, wx)
    wys = (wy2, wy)
    wzs = (wz2, wz)
    weights = []
    for dx in range(2):
        for dy in range(2):
            wxy = wxs[dx] * wys[dy]
            for dz in range(2):
                weights.append(wxy * wzs[dz])
    return weights


T0 = B * S0 * S0 * (S0 // 8)
T1 = B * S1 * S1 * (S1 // 8)
NT = 32


def _depad_one(src, dst, vb, cb, isem, osem, nt_total, wid):
    tbase = wid * (nt_total // NW)
    nsteps = nt_total // NW // NT

    def fire_in(i, bf):
        for s in range(8):
            pltpu.async_copy(
                src.at[pl.ds(tbase + i * NT, NT), pl.ds(s, 1)],
                vb[bf].at[s], isem[bf])

    def wait_in(i, bf):
        for s in range(8):
            pltpu.make_async_copy(
                src.at[pl.ds(tbase + i * NT, NT), pl.ds(s, 1)],
                vb[bf].at[s], isem[bf]).wait()

    def out_slice(i):
        return dst.at[pl.ds((tbase + i * NT) * 256, NT * 256)]

    fire_in(0, 0)
    fire_in(1, 1)

    @pl.loop(0, nsteps, step=2)
    def _steps(i):
        for bf in range(2):
            cur = i + bf
            wait_in(cur, bf)

            @pl.when(cur >= 2)
            def _():
                pltpu.make_async_copy(cb[bf], out_slice(cur), osem[bf]).wait()

            for t in range(NT):
                for s in range(8):
                    o = (t * 8 + s) * 32
                    cb[bf][pl.ds(o, L)] = vb[bf][s, t, 0, pl.ds(0, L)]
                    cb[bf][pl.ds(o + L, L)] = vb[bf][s, t, 0, pl.ds(L, L)]
            pltpu.async_copy(cb[bf], out_slice(cur), osem[bf])

            @pl.when(cur + 2 < nsteps)
            def _():
                fire_in(cur + 2, bf)

    for bf in range(2):
        pltpu.make_async_copy(
            cb[bf], out_slice(nsteps - 2 + bf), osem[bf]).wait()


def _depad_body(a0, a1, out0, out1, vb0, vb1, cb0, cb1,
                isem0, isem1, osem0, osem1):
    cid = lax.axis_index("c")
    sid = lax.axis_index("s")
    wid = sid * NC + cid
    vb = (vb0, vb1)
    cb = (cb0, cb1)
    isem = (isem0, isem1)
    osem = (osem0, osem1)
    _depad_one(a0, out0, vb, cb, isem, osem, T0, wid)
    _depad_one(a1, out1, vb, cb, isem, osem, T1, wid)


def _body(f0, f1, coords, out,
          cv, idx0a, idx0b, idx1a, idx1b,
          rows0a, rows0b, rows1a, rows1b, outv, sema, semb):
    cid = lax.axis_index("c")
    sid = lax.axis_index("s")
    wid = sid * NC + cid
    base = wid * PPW
    bb = lax.shift_right_logical(base, 13)
    m0 = base - bb * M

    pltpu.sync_copy(coords.at[bb, pl.ds(m0, PPW)], cv)

    lane = lax.iota(jnp.int32, L)
    zero = jnp.zeros((L,), jnp.int32)
    one = zero + 1
    two = zero + 2
    bbase0 = lax.shift_left(bb, 18)
    bbase1 = lax.shift_left(bb, 15)

    idx0 = (idx0a, idx0b)
    idx1 = (idx1a, idx1b)
    rows0 = (rows0a, rows0b)
    rows1 = (rows1a, rows1b)
    sem = (sema, semb)

    def load_xyz(ci):
        pt = lane + ci * CHUNK
        px = plsc.load_gather(cv, [pt, zero])
        py = plsc.load_gather(cv, [pt, one])
        pz = plsc.load_gather(cv, [pt, two])
        return px, py, pz

    def fire(ci, bf):
        px, py, pz = load_xyz(ci)
        r0 = _corner_rows(px, py, pz, float(S0), CAP0, bbase0, S0)
        r1 = _corner_rows(px, py, pz, float(S1), CAP1, bbase1, S1)
        for k in range(8):
            idx0[bf][pl.ds(k * L, L)] = r0[k]
            idx1[bf][pl.ds(k * L, L)] = r1[k]
        pltpu.async_copy(f0.at[idx0[bf]], rows0[bf], sem[bf])
        pltpu.async_copy(f1.at[idx1[bf]], rows1[bf], sem[bf])

    def compute(ci, bf):
        px, py, pz = load_xyz(ci)
        w0 = _corner_weights(px, py, pz, float(S0), CAP0)
        w1 = _corner_weights(px, py, pz, float(S1), CAP1)
        pltpu.make_async_copy(f0.at[idx0[bf]], rows0[bf], sem[bf]).wait()
        pltpu.make_async_copy(f1.at[idx1[bf]], rows1[bf], sem[bf]).wait()

        p0 = ci * CHUNK
        for p in range(CHUNK):
            for rows, w8, choff in ((rows0[bf], w0, 0), (rows1[bf], w1, C0)):
                w = jnp.broadcast_to(w8[0][p], (L,))
                acc_a = rows[p, pl.ds(0, L)] * w
                acc_b = rows[p, pl.ds(L, L)] * w
                for k in range(1, 8):
                    w = jnp.broadcast_to(w8[k][p], (L,))
                    acc_a = acc_a + rows[k * L + p, pl.ds(0, L)] * w
                    acc_b = acc_b + rows[k * L + p, pl.ds(L, L)] * w
                outv[p0 + p, pl.ds(choff, L)] = acc_a
                outv[p0 + p, pl.ds(choff + L, L)] = acc_b

    fire(0, 0)
    fire(1, 1)

    @pl.loop(0, NCHUNK, step=2)
    def _chunks(ci):
        for bf in range(2):
            cur = ci + bf
            compute(cur, bf)
            nxt = cur + 2

            @pl.when(nxt < NCHUNK)
            def _():
                fire(nxt, bf)

    pltpu.sync_copy(outv, out.at[pl.ds(base, PPW)])


@jax.jit
def kernel(feat0, feat1, mesh_coords):
    mesh = plsc.VectorSubcoreMesh(
        core_axis_name="c", subcore_axis_name="s",
        num_cores=NC, num_subcores=NS)

    depad = pl.kernel(
        _depad_body,
        out_type=(jax.ShapeDtypeStruct((T0 * 256,), jnp.float32),
                  jax.ShapeDtypeStruct((T1 * 256,), jnp.float32)),
        mesh=mesh,
        scratch_types=[
            pltpu.VMEM((8, NT, 1, C0), jnp.float32),
            pltpu.VMEM((8, NT, 1, C0), jnp.float32),
            pltpu.VMEM((NT * 256,), jnp.float32),
            pltpu.VMEM((NT * 256,), jnp.float32),
            pltpu.SemaphoreType.DMA,
            pltpu.SemaphoreType.DMA,
            pltpu.SemaphoreType.DMA,
            pltpu.SemaphoreType.DMA,
        ],
        compiler_params=pltpu.CompilerParams(
            needs_layout_passes=False, use_tc_tiling_on_sc=True),
    )
    d0, d1 = depad(feat0.reshape(T0, 8, C0), feat1.reshape(T1, 8, C0))
    f0 = d0.reshape(B * S0 * S0 * S0, C0)
    f1 = d1.reshape(B * S1 * S1 * S1, C0)
    run = pl.kernel(
        _body,
        out_type=jax.ShapeDtypeStruct((N, 2 * C0), jnp.float32),
        mesh=mesh,
        scratch_types=[
            pltpu.VMEM((PPW, 3), jnp.float32),
            pltpu.VMEM((8 * L,), jnp.int32),
            pltpu.VMEM((8 * L,), jnp.int32),
            pltpu.VMEM((8 * L,), jnp.int32),
            pltpu.VMEM((8 * L,), jnp.int32),
            pltpu.VMEM((8 * L, C0), jnp.float32),
            pltpu.VMEM((8 * L, C0), jnp.float32),
            pltpu.VMEM((8 * L, C0), jnp.float32),
            pltpu.VMEM((8 * L, C0), jnp.float32),
            pltpu.VMEM((PPW, 2 * C0), jnp.float32),
            pltpu.SemaphoreType.DMA,
            pltpu.SemaphoreType.DMA,
        ],
        compiler_params=pltpu.CompilerParams(
            needs_layout_passes=False, use_tc_tiling_on_sc=False),
    )
    out = run(f0, f1, mesh_coords)
    return out.reshape(B, M, 2 * C0)


# speedup vs baseline: 1.0065x; 1.0065x over previous
import jax
import jax.numpy as jnp
from jax import lax
from jax.experimental import pallas as pl
from jax.experimental.pallas import tpu as pltpu
from jax.experimental.pallas import tpu_sc as plsc

NC = 2
NS = 16
NW = NC * NS
L = 16

B = 4
M = 8192
N = B * M
PPW = N // NW
CHUNK = 16
NCHUNK = PPW // CHUNK

C0 = 32
S0, S1 = 64, 32
CAP0 = float(S0 - 1.01)
CAP1 = float(S1 - 1.01)
LO = 0.01


def _axis(p, scale, cap):
    t = jnp.clip(p * scale, LO, cap)
    i1 = t.astype(jnp.int32)
    f1 = i1.astype(jnp.float32)
    i2 = i1 + jnp.where(t != f1, 1, 0)
    f2 = i2.astype(jnp.float32)
    return i1, i2, t - f1, f2 - t


def _corner_rows(px, py, pz, scale, cap, bbase, s):
    x1, x2, _, _ = _axis(px, scale, cap)
    y1, y2, _, _ = _axis(py, scale, cap)
    z1, z2, _, _ = _axis(pz, scale, cap)
    s2 = s * s
    ax = (bbase + x1 * s2, bbase + x2 * s2)
    ay = (y1 * s, y2 * s)
    az = (z1, z2)
    rows = []
    for dx in range(2):
        for dy in range(2):
            axy = ax[dx] + ay[dy]
            for dz in range(2):
                rows.append(axy + az[dz])
    return rows


def _corner_weights(px, py, pz, scale, cap):
    _, _, wx, wx2 = _axis(px, scale, cap)
    _, _, wy, wy2 = _axis(py, scale, cap)
    _, _, wz, wz2 = _axis(pz, scale, cap)
    wxs = (wx2, wx)
    wys = (wy2, wy)
    wzs = (wz2, wz)
    weights = []
    for dx in range(2):
        for dy in range(2):
            wxy = wxs[dx] * wys[dy]
            for dz in range(2):
                weights.append(wxy * wzs[dz])
    return weights


T0 = B * S0 * S0 * (S0 // 8)
T1 = B * S1 * S1 * (S1 // 8)
NT = 32


def _depad_one(src, dst, vb, cb, isem, osem, nt_total, wid):
    tbase = wid * (nt_total // NW)
    nsteps = nt_total // NW // NT

    def fire_in(i, bf):
        pltpu.async_copy(src.at[pl.ds(tbase + i * NT, NT)], vb[bf], isem[bf])

    def wait_in(i, bf):
        pltpu.make_async_copy(
            src.at[pl.ds(tbase + i * NT, NT)], vb[bf], isem[bf]).wait()

    def out_slice(i):
        return dst.at[pl.ds((tbase + i * NT) * 2, NT * 2)]

    fire_in(0, 0)
    fire_in(1, 1)

    @pl.loop(0, nsteps, step=2)
    def _steps(i):
        for bf in range(2):
            cur = i + bf
            wait_in(cur, bf)

            @pl.when(cur >= 2)
            def _():
                pltpu.make_async_copy(cb[bf], out_slice(cur), osem[bf]).wait()

            for t in range(NT):
                for s in range(8):
                    r = t * 8 + s
                    o = (r % 4) * 32
                    cb[bf][r // 4, pl.ds(o, L)] = vb[bf][t, s, pl.ds(0, L)]
                    cb[bf][r // 4, pl.ds(o + L, L)] = vb[bf][t, s, pl.ds(L, L)]
            pltpu.async_copy(cb[bf], out_slice(cur), osem[bf])

            @pl.when(cur + 2 < nsteps)
            def _():
                fire_in(cur + 2, bf)

    for bf in range(2):
        pltpu.make_async_copy(
            cb[bf], out_slice(nsteps - 2 + bf), osem[bf]).wait()


def _depad_body(a0, a1, out0, out1, vb0, vb1, cb0, cb1,
                isem0, isem1, osem0, osem1):
    cid = lax.axis_index("c")
    sid = lax.axis_index("s")
    wid = sid * NC + cid
    vb = (vb0, vb1)
    cb = (cb0, cb1)
    isem = (isem0, isem1)
    osem = (osem0, osem1)
    _depad_one(a0, out0, vb, cb, isem, osem, T0, wid)
    _depad_one(a1, out1, vb, cb, isem, osem, T1, wid)


def _body(f0, f1, coords, out,
          cv, idx0a, idx0b, idx1a, idx1b,
          rows0a, rows0b, rows1a, rows1b, outv, sema, semb):
    cid = lax.axis_index("c")
    sid = lax.axis_index("s")
    wid = sid * NC + cid
    base = wid * PPW
    bb = lax.shift_right_logical(base, 13)
    m0 = base - bb * M

    pltpu.sync_copy(coords.at[bb, pl.ds(m0, PPW)], cv)

    lane = lax.iota(jnp.int32, L)
    zero = jnp.zeros((L,), jnp.int32)
    one = zero + 1
    two = zero + 2
    bbase0 = lax.shift_left(bb, 18)
    bbase1 = lax.shift_left(bb, 15)

    idx0 = (idx0a, idx0b)
    idx1 = (idx1a, idx1b)
    rows0 = (rows0a, rows0b)
    rows1 = (rows1a, rows1b)
    sem = (sema, semb)

    def load_xyz(ci):
        pt = lane + ci * CHUNK
        px = plsc.load_gather(cv, [pt, zero])
        py = plsc.load_gather(cv, [pt, one])
        pz = plsc.load_gather(cv, [pt, two])
        return px, py, pz

    def fire(ci, bf):
        px, py, pz = load_xyz(ci)
        r0 = _corner_rows(px, py, pz, float(S0), CAP0, bbase0, S0)
        r1 = _corner_rows(px, py, pz, float(S1), CAP1, bbase1, S1)
        for k in range(8):
            idx0[bf][pl.ds(k * L, L)] = r0[k]
            idx1[bf][pl.ds(k * L, L)] = r1[k]
        pltpu.async_copy(f0.at[idx0[bf]], rows0[bf], sem[bf])
        pltpu.async_copy(f1.at[idx1[bf]], rows1[bf], sem[bf])

    def compute(ci, bf):
        px, py, pz = load_xyz(ci)
        w0 = _corner_weights(px, py, pz, float(S0), CAP0)
        w1 = _corner_weights(px, py, pz, float(S1), CAP1)
        pltpu.make_async_copy(f0.at[idx0[bf]], rows0[bf], sem[bf]).wait()
        pltpu.make_async_copy(f1.at[idx1[bf]], rows1[bf], sem[bf]).wait()

        p0 = ci * CHUNK
        for p in range(CHUNK):
            for rows, w8, choff in ((rows0[bf], w0, 0), (rows1[bf], w1, C0)):
                w = jnp.broadcast_to(w8[0][p], (L,))
                acc_a = rows[p, pl.ds(0, L)] * w
                acc_b = rows[p, pl.ds(L, L)] * w
                for k in range(1, 8):
                    w = jnp.broadcast_to(w8[k][p], (L,))
                    acc_a = acc_a + rows[k * L + p, pl.ds(0, L)] * w
                    acc_b = acc_b + rows[k * L + p, pl.ds(L, L)] * w
                outv[p0 + p, pl.ds(choff, L)] = acc_a
                outv[p0 + p, pl.ds(choff + L, L)] = acc_b

    fire(0, 0)
    fire(1, 1)

    @pl.loop(0, NCHUNK, step=2)
    def _chunks(ci):
        for bf in range(2):
            cur = ci + bf
            compute(cur, bf)
            nxt = cur + 2

            @pl.when(nxt < NCHUNK)
            def _():
                fire(nxt, bf)

    pltpu.sync_copy(outv, out.at[pl.ds(base, PPW)])


@jax.jit
def kernel(feat0, feat1, mesh_coords):
    mesh = plsc.VectorSubcoreMesh(
        core_axis_name="c", subcore_axis_name="s",
        num_cores=NC, num_subcores=NS)

    depad = pl.kernel(
        _depad_body,
        out_type=(jax.ShapeDtypeStruct((T0 * 2, 128), jnp.float32),
                  jax.ShapeDtypeStruct((T1 * 2, 128), jnp.float32)),
        mesh=mesh,
        scratch_types=[
            pltpu.VMEM((NT, 8, C0), jnp.float32),
            pltpu.VMEM((NT, 8, C0), jnp.float32),
            pltpu.VMEM((NT * 2, 128), jnp.float32),
            pltpu.VMEM((NT * 2, 128), jnp.float32),
            pltpu.SemaphoreType.DMA,
            pltpu.SemaphoreType.DMA,
            pltpu.SemaphoreType.DMA,
            pltpu.SemaphoreType.DMA,
        ],
        compiler_params=pltpu.CompilerParams(
            needs_layout_passes=False, use_tc_tiling_on_sc=True),
    )
    d0, d1 = depad(feat0.reshape(T0, 8, C0), feat1.reshape(T1, 8, C0))
    f0 = d0.reshape(B * S0 * S0 * S0, C0)
    f1 = d1.reshape(B * S1 * S1 * S1, C0)
    run = pl.kernel(
        _body,
        out_type=jax.ShapeDtypeStruct((N, 2 * C0), jnp.float32),
        mesh=mesh,
        scratch_types=[
            pltpu.VMEM((PPW, 3), jnp.float32),
            pltpu.VMEM((8 * L,), jnp.int32),
            pltpu.VMEM((8 * L,), jnp.int32),
            pltpu.VMEM((8 * L,), jnp.int32),
            pltpu.VMEM((8 * L,), jnp.int32),
            pltpu.VMEM((8 * L, C0), jnp.float32),
            pltpu.VMEM((8 * L, C0), jnp.float32),
            pltpu.VMEM((8 * L, C0), jnp.float32),
            pltpu.VMEM((8 * L, C0), jnp.float32),
            pltpu.VMEM((PPW, 2 * C0), jnp.float32),
            pltpu.SemaphoreType.DMA,
            pltpu.SemaphoreType.DMA,
        ],
        compiler_params=pltpu.CompilerParams(
            needs_layout_passes=False, use_tc_tiling_on_sc=False),
    )
    out = run(f0, f1, mesh_coords)
    return out.reshape(B, M, 2 * C0)
